# baseline (speedup 1.0000x reference)
import functools

import jax
import jax.numpy as jnp
from jax import lax
from jax.experimental import pallas as pl
from jax.experimental.pallas import tpu as pltpu
from jax.experimental.pallas import tpu_sc as plsc

B = 1024
L = 50
E = 64
H = 64
VHALF = 500000
NW = 32
N_IDX = B * L
B_PER_W = N_IDX // NW
CHUNK = 80
NCHUNK = B_PER_W // CHUNK
NPASS = 2
CPP = NCHUNK // NPASS

T_BLK = 2
N_BLK = L // T_BLK


def _pack_body(in_ref, eye_ref, out_ref):
    x = in_ref[...]
    xT = jax.lax.dot_general(
        x, eye_ref[...], (((0,), (0,)), ((), ())),
        preferred_element_type=jnp.float32,
    )
    out_ref[...] = jnp.concatenate([xT[0:1024, :], xT[1024:2048, :]], axis=1)


def _pack(embT, eye):
    return pl.pallas_call(
        _pack_body,
        grid=(489,),
        in_specs=[
            pl.BlockSpec((E, 2048), lambda i: (0, i)),
            pl.BlockSpec((E, E), lambda i: (0, 0)),
        ],
        out_specs=pl.BlockSpec((1024, 2 * E), lambda i: (i, 0)),
        out_shape=jax.ShapeDtypeStruct((489 * 1024, 2 * E), jnp.float32),
        compiler_params=pltpu.CompilerParams(
            dimension_semantics=("arbitrary",),
        ),
    )(embT, eye)


def _sc_gather_body(table_hbm, idx_hbm, out_hbm, idx_v, rows_v, sem):
    wid = lax.axis_index("s") * 2 + lax.axis_index("c")
    base = wid * B_PER_W
    pltpu.sync_copy(idx_hbm.at[wid], idx_v)
    for p in range(NPASS):
        copies = []
        for j in range(CPP):
            copies.append(
                pltpu.async_copy(
                    table_hbm.at[idx_v.at[p * CPP + j]],
                    rows_v.at[pl.ds(j * CHUNK, CHUNK)],
                    sem,
                )
            )
        for cp in copies:
            cp.wait()
        pltpu.sync_copy(
            rows_v, out_hbm.at[pl.ds(base + p * CPP * CHUNK, CPP * CHUNK)]
        )


def _sc_gather(table, idx3):
    kern = functools.partial(
        pl.kernel,
        mesh=plsc.VectorSubcoreMesh(core_axis_name="c", subcore_axis_name="s"),
        out_type=jax.ShapeDtypeStruct((N_IDX, 2 * E), jnp.float32),
        scratch_types=[
            pltpu.VMEM((NCHUNK, CHUNK), jnp.int32),
            pltpu.VMEM((CPP * CHUNK, 2 * E), jnp.float32),
            pltpu.SemaphoreType.DMA,
        ],
    )(_sc_gather_body)
    return kern(table, idx3)


def _lstm_body(x_ref, sel_ref, w_ref, b_ref, h0_ref, c0_ref,
               ys_ref, h_ref, c_ref):
    i = pl.program_id(0)

    @pl.when(i == 0)
    def _():
        h_ref[...] = h0_ref[...]
        c_ref[...] = c0_ref[...]

    h = h_ref[...]
    c = c_ref[...]
    b = b_ref[...]
    w = w_ref[...]
    for j in range(T_BLK):
        xf = x_ref[j * B:(j + 1) * B, :]
        sel = sel_ref[j * B:(j + 1) * B, :] != 0
        x_t = jnp.where(sel, xf[:, E:2 * E], xf[:, 0:E])
        xh = jnp.concatenate([x_t, h], axis=1)
        gates = jnp.dot(xh, w, preferred_element_type=jnp.float32) + b
        ig = jax.nn.sigmoid(gates[:, 0:H])
        fg = jax.nn.sigmoid(gates[:, H:2 * H])
        gg = jnp.tanh(gates[:, 2 * H:3 * H])
        og = jax.nn.sigmoid(gates[:, 3 * H:4 * H])
        c = fg * c + ig * gg
        h = og * jnp.tanh(c)
        ys_ref[:, j * H:(j + 1) * H] = h
    h_ref[...] = h
    c_ref[...] = c


def _lstm(x, sel2d, w_cat, bias, h0, c0, interpret=False):
    return pl.pallas_call(
        _lstm_body,
        grid=(N_BLK,),
        in_specs=[
            pl.BlockSpec((T_BLK * B, 2 * E), lambda i: (i, 0)),
            pl.BlockSpec((T_BLK * B, E), lambda i: (i, 0)),
            pl.BlockSpec((E + H, 4 * H), lambda i: (0, 0)),
            pl.BlockSpec((1, 4 * H), lambda i: (0, 0)),
            pl.BlockSpec((B, H), lambda i: (0, 0)),
            pl.BlockSpec((B, H), lambda i: (0, 0)),
        ],
        out_specs=[
            pl.BlockSpec((B, T_BLK * H), lambda i: (0, i)),
            pl.BlockSpec((B, H), lambda i: (0, 0)),
            pl.BlockSpec((B, H), lambda i: (0, 0)),
        ],
        out_shape=[
            jax.ShapeDtypeStruct((B, L * H), jnp.float32),
            jax.ShapeDtypeStruct((B, H), jnp.float32),
            jax.ShapeDtypeStruct((B, H), jnp.float32),
        ],
        compiler_params=pltpu.CompilerParams(
            dimension_semantics=("arbitrary",),
        ),
        interpret=interpret,
    )(x, sel2d, w_cat, bias, h0, c0)


def kernel(decoder_input, h0, c0, emb, W_ih, W_hh, b_ih, b_hh):
    idxT = decoder_input.T.reshape(-1).astype(jnp.int32)
    selT = ((idxT >> 10) & 1).astype(jnp.int8)
    idx_p = ((idxT >> 11) << 10) | (idxT & 1023)
    idx3 = idx_p.reshape(NW, NCHUNK, CHUNK)
    table = _pack(emb.T, jnp.eye(E, dtype=jnp.float32))
    x = _sc_gather(table, idx3)
    sel2d = jnp.broadcast_to(selT.reshape(N_IDX, 1), (N_IDX, E))
    w_cat = jnp.concatenate([W_ih.T, W_hh.T], axis=0)
    bias = (b_ih + b_hh).reshape(1, 4 * H)
    ys2d, h_n, c_n = _lstm(x, sel2d, w_cat, bias, h0[0], c0[0])
    decoder_output = ys2d.reshape(B, L, H)
    return decoder_output, (h_n[None, :, :], c_n[None, :, :])


# speedup vs baseline: 1.5177x; 1.5177x over previous
import functools

import jax
import jax.numpy as jnp
from jax import lax
from jax.experimental import pallas as pl
from jax.experimental.pallas import tpu as pltpu
from jax.experimental.pallas import tpu_sc as plsc

B = 1024
L = 50
E = 64
H = 64
VHALF = 500000
NW = 32
N_IDX = B * L
B_PER_W = N_IDX // NW
CHUNK = 80
NCHUNK = B_PER_W // CHUNK
NPASS = 2
CPP = NCHUNK // NPASS

T_BLK = 2
N_BLK = L // T_BLK


PBLK = 8192
NPBLK = 123


def _pack_body(in_ref, out_ref):
    x = in_ref[...]
    xT = x.T
    out_ref[...] = jnp.concatenate(
        [xT[0:PBLK // 2, :], xT[PBLK // 2:PBLK, :]], axis=1
    )


def _pack(embT):
    return pl.pallas_call(
        _pack_body,
        grid=(NPBLK,),
        in_specs=[pl.BlockSpec((E, PBLK), lambda i: (0, i))],
        out_specs=pl.BlockSpec((PBLK // 2, 2 * E), lambda i: (i, 0)),
        out_shape=jax.ShapeDtypeStruct((NPBLK * PBLK // 2, 2 * E), jnp.float32),
        compiler_params=pltpu.CompilerParams(
            dimension_semantics=("parallel",),
        ),
    )(embT)


def _sc_gather_body(table_hbm, idx_hbm, out_hbm, idx_v, rows_v, sem):
    wid = lax.axis_index("s") * 2 + lax.axis_index("c")
    base = wid * B_PER_W
    pltpu.sync_copy(idx_hbm.at[wid], idx_v)
    for p in range(NPASS):
        copies = []
        for j in range(CPP):
            copies.append(
                pltpu.async_copy(
                    table_hbm.at[idx_v.at[p * CPP + j]],
                    rows_v.at[pl.ds(j * CHUNK, CHUNK)],
                    sem,
                )
            )
        for cp in copies:
            cp.wait()
        pltpu.sync_copy(
            rows_v, out_hbm.at[pl.ds(base + p * CPP * CHUNK, CPP * CHUNK)]
        )


def _sc_gather(table, idx3):
    kern = functools.partial(
        pl.kernel,
        mesh=plsc.VectorSubcoreMesh(core_axis_name="c", subcore_axis_name="s"),
        out_type=jax.ShapeDtypeStruct((N_IDX, 2 * E), jnp.float32),
        scratch_types=[
            pltpu.VMEM((NCHUNK, CHUNK), jnp.int32),
            pltpu.VMEM((CPP * CHUNK, 2 * E), jnp.float32),
            pltpu.SemaphoreType.DMA,
        ],
    )(_sc_gather_body)
    return kern(table, idx3)


def _lstm_body(x_ref, sel_ref, w_ref, b_ref, h0_ref, c0_ref,
               ys_ref, h_ref, c_ref):
    i = pl.program_id(0)

    @pl.when(i == 0)
    def _():
        h_ref[...] = h0_ref[...]
        c_ref[...] = c0_ref[...]

    h = h_ref[...]
    c = c_ref[...]
    b = b_ref[...]
    w = w_ref[...]
    for j in range(T_BLK):
        xf = x_ref[j * B:(j + 1) * B, :]
        sel = sel_ref[j * B:(j + 1) * B, :] != 0
        x_t = jnp.where(sel, xf[:, E:2 * E], xf[:, 0:E])
        xh = jnp.concatenate([x_t, h], axis=1)
        gates = jnp.dot(xh, w, preferred_element_type=jnp.float32) + b
        ig = jax.nn.sigmoid(gates[:, 0:H])
        fg = jax.nn.sigmoid(gates[:, H:2 * H])
        gg = jnp.tanh(gates[:, 2 * H:3 * H])
        og = jax.nn.sigmoid(gates[:, 3 * H:4 * H])
        c = fg * c + ig * gg
        h = og * jnp.tanh(c)
        ys_ref[:, j * H:(j + 1) * H] = h
    h_ref[...] = h
    c_ref[...] = c


def _lstm(x, sel2d, w_cat, bias, h0, c0, interpret=False):
    return pl.pallas_call(
        _lstm_body,
        grid=(N_BLK,),
        in_specs=[
            pl.BlockSpec((T_BLK * B, 2 * E), lambda i: (i, 0)),
            pl.BlockSpec((T_BLK * B, E), lambda i: (i, 0)),
            pl.BlockSpec((E + H, 4 * H), lambda i: (0, 0)),
            pl.BlockSpec((1, 4 * H), lambda i: (0, 0)),
            pl.BlockSpec((B, H), lambda i: (0, 0)),
            pl.BlockSpec((B, H), lambda i: (0, 0)),
        ],
        out_specs=[
            pl.BlockSpec((B, T_BLK * H), lambda i: (0, i)),
            pl.BlockSpec((B, H), lambda i: (0, 0)),
            pl.BlockSpec((B, H), lambda i: (0, 0)),
        ],
        out_shape=[
            jax.ShapeDtypeStruct((B, L * H), jnp.float32),
            jax.ShapeDtypeStruct((B, H), jnp.float32),
            jax.ShapeDtypeStruct((B, H), jnp.float32),
        ],
        compiler_params=pltpu.CompilerParams(
            dimension_semantics=("arbitrary",),
        ),
        interpret=interpret,
    )(x, sel2d, w_cat, bias, h0, c0)


def kernel(decoder_input, h0, c0, emb, W_ih, W_hh, b_ih, b_hh):
    idxT = decoder_input.T.reshape(-1).astype(jnp.int32)
    selT = ((idxT >> 12) & 1).astype(jnp.int8)
    idx_p = ((idxT >> 13) << 12) | (idxT & 4095)
    idx3 = idx_p.reshape(NW, NCHUNK, CHUNK)
    table = _pack(emb.T)
    x = _sc_gather(table, idx3)
    sel2d = jnp.broadcast_to(selT.reshape(N_IDX, 1), (N_IDX, E))
    w_cat = jnp.concatenate([W_ih.T, W_hh.T], axis=0)
    bias = (b_ih + b_hh).reshape(1, 4 * H)
    ys2d, h_n, c_n = _lstm(x, sel2d, w_cat, bias, h0[0], c0[0])
    decoder_output = ys2d.reshape(B, L, H)
    return decoder_output, (h_n[None, :, :], c_n[None, :, :])


# speedup vs baseline: 1.6562x; 1.0913x over previous
import functools

import jax
import jax.numpy as jnp
from jax import lax
from jax.experimental import pallas as pl
from jax.experimental.pallas import tpu as pltpu
from jax.experimental.pallas import tpu_sc as plsc

B = 1024
L = 50
E = 64
H = 64
VHALF = 500000
NW = 32
N_IDX = B * L
B_PER_W = N_IDX // NW
CHUNK = 80
NCHUNK = B_PER_W // CHUNK
NPASS = 2
CPP = NCHUNK // NPASS

T_BLK = 2
N_BLK = L // T_BLK


PBLK = 16384
NPBLK = 62


def _pack_body(in_ref, out_ref):
    x = in_ref[...]
    xT = x.T
    out_ref[...] = jnp.concatenate(
        [xT[0:PBLK // 2, :], xT[PBLK // 2:PBLK, :]], axis=1
    )


def _pack(embT):
    return pl.pallas_call(
        _pack_body,
        grid=(NPBLK,),
        in_specs=[pl.BlockSpec((E, PBLK), lambda i: (0, i))],
        out_specs=pl.BlockSpec((PBLK // 2, 2 * E), lambda i: (i, 0)),
        out_shape=jax.ShapeDtypeStruct((NPBLK * PBLK // 2, 2 * E), jnp.float32),
        compiler_params=pltpu.CompilerParams(
            dimension_semantics=("parallel",),
        ),
    )(embT)


def _sc_gather_body(table_hbm, idx_hbm, out_hbm, idx_v, rows_v, sem):
    wid = lax.axis_index("s") * 2 + lax.axis_index("c")
    base = wid * B_PER_W
    pltpu.sync_copy(idx_hbm.at[wid], idx_v)
    for p in range(NPASS):
        copies = []
        for j in range(CPP):
            copies.append(
                pltpu.async_copy(
                    table_hbm.at[idx_v.at[p * CPP + j]],
                    rows_v.at[pl.ds(j * CHUNK, CHUNK)],
                    sem,
                )
            )
        for cp in copies:
            cp.wait()
        pltpu.sync_copy(
            rows_v, out_hbm.at[pl.ds(base + p * CPP * CHUNK, CPP * CHUNK)]
        )


def _sc_gather(table, idx3):
    kern = functools.partial(
        pl.kernel,
        mesh=plsc.VectorSubcoreMesh(core_axis_name="c", subcore_axis_name="s"),
        out_type=jax.ShapeDtypeStruct((N_IDX, 2 * E), jnp.float32),
        scratch_types=[
            pltpu.VMEM((NCHUNK, CHUNK), jnp.int32),
            pltpu.VMEM((CPP * CHUNK, 2 * E), jnp.float32),
            pltpu.SemaphoreType.DMA,
        ],
    )(_sc_gather_body)
    return kern(table, idx3)


def _lstm_body(x_ref, sel_ref, w_ref, b_ref, h0_ref, c0_ref,
               ys_ref, h_ref, c_ref):
    i = pl.program_id(0)

    @pl.when(i == 0)
    def _():
        h_ref[...] = h0_ref[...]
        c_ref[...] = c0_ref[...]

    h = h_ref[...]
    c = c_ref[...]
    b = b_ref[...]
    w = w_ref[...]
    for j in range(T_BLK):
        xf = x_ref[j * B:(j + 1) * B, :]
        sel = sel_ref[j * B:(j + 1) * B, :] != 0
        x_t = jnp.where(sel, xf[:, E:2 * E], xf[:, 0:E])
        xh = jnp.concatenate([x_t, h], axis=1)
        gates = jnp.dot(xh, w, preferred_element_type=jnp.float32) + b
        ig = jax.nn.sigmoid(gates[:, 0:H])
        fg = jax.nn.sigmoid(gates[:, H:2 * H])
        gg = jnp.tanh(gates[:, 2 * H:3 * H])
        og = jax.nn.sigmoid(gates[:, 3 * H:4 * H])
        c = fg * c + ig * gg
        h = og * jnp.tanh(c)
        ys_ref[:, j * H:(j + 1) * H] = h
    h_ref[...] = h
    c_ref[...] = c


def _lstm(x, sel2d, w_cat, bias, h0, c0, interpret=False):
    return pl.pallas_call(
        _lstm_body,
        grid=(N_BLK,),
        in_specs=[
            pl.BlockSpec((T_BLK * B, 2 * E), lambda i: (i, 0)),
            pl.BlockSpec((T_BLK * B, E), lambda i: (i, 0)),
            pl.BlockSpec((E + H, 4 * H), lambda i: (0, 0)),
            pl.BlockSpec((1, 4 * H), lambda i: (0, 0)),
            pl.BlockSpec((B, H), lambda i: (0, 0)),
            pl.BlockSpec((B, H), lambda i: (0, 0)),
        ],
        out_specs=[
            pl.BlockSpec((B, T_BLK * H), lambda i: (0, i)),
            pl.BlockSpec((B, H), lambda i: (0, 0)),
            pl.BlockSpec((B, H), lambda i: (0, 0)),
        ],
        out_shape=[
            jax.ShapeDtypeStruct((B, L * H), jnp.float32),
            jax.ShapeDtypeStruct((B, H), jnp.float32),
            jax.ShapeDtypeStruct((B, H), jnp.float32),
        ],
        compiler_params=pltpu.CompilerParams(
            dimension_semantics=("arbitrary",),
        ),
        interpret=interpret,
    )(x, sel2d, w_cat, bias, h0, c0)


def kernel(decoder_input, h0, c0, emb, W_ih, W_hh, b_ih, b_hh):
    idxT = decoder_input.T.reshape(-1).astype(jnp.int32)
    selT = ((idxT >> 13) & 1).astype(jnp.int8)
    idx_p = ((idxT >> 14) << 13) | (idxT & 8191)
    idx3 = idx_p.reshape(NW, NCHUNK, CHUNK)
    table = _pack(emb.T)
    x = _sc_gather(table, idx3)
    sel2d = jnp.broadcast_to(selT.reshape(N_IDX, 1), (N_IDX, E))
    w_cat = jnp.concatenate([W_ih.T, W_hh.T], axis=0)
    bias = (b_ih + b_hh).reshape(1, 4 * H)
    ys2d, h_n, c_n = _lstm(x, sel2d, w_cat, bias, h0[0], c0[0])
    decoder_output = ys2d.reshape(B, L, H)
    return decoder_output, (h_n[None, :, :], c_n[None, :, :])


# speedup vs baseline: 1.7194x; 1.0381x over previous
import functools

import jax
import jax.numpy as jnp
from jax import lax
from jax.experimental import pallas as pl
from jax.experimental.pallas import tpu as pltpu
from jax.experimental.pallas import tpu_sc as plsc

B = 1024
L = 50
E = 64
H = 64
VHALF = 500000
NW = 32
N_IDX = B * L
B_PER_W = N_IDX // NW
CHUNK = 80
NCHUNK = B_PER_W // CHUNK
NPASS = 2
CPP = NCHUNK // NPASS

T_BLK = 2
N_BLK = L // T_BLK


PBLK = 32768
NPBLK = 31


def _pack_body(in_ref, out_ref):
    x = in_ref[...]
    xT = x.T
    out_ref[...] = jnp.concatenate(
        [xT[0:PBLK // 2, :], xT[PBLK // 2:PBLK, :]], axis=1
    )


def _pack(embT):
    return pl.pallas_call(
        _pack_body,
        grid=(NPBLK,),
        in_specs=[pl.BlockSpec((E, PBLK), lambda i: (0, i))],
        out_specs=pl.BlockSpec((PBLK // 2, 2 * E), lambda i: (i, 0)),
        out_shape=jax.ShapeDtypeStruct((NPBLK * PBLK // 2, 2 * E), jnp.float32),
        compiler_params=pltpu.CompilerParams(
            dimension_semantics=("parallel",),
        ),
    )(embT)


def _sc_gather_body(table_hbm, idx_hbm, out_hbm, idx_v, rows_v, sem):
    wid = lax.axis_index("s") * 2 + lax.axis_index("c")
    base = wid * B_PER_W
    pltpu.sync_copy(idx_hbm.at[wid], idx_v)
    for p in range(NPASS):
        copies = []
        for j in range(CPP):
            copies.append(
                pltpu.async_copy(
                    table_hbm.at[idx_v.at[p * CPP + j]],
                    rows_v.at[pl.ds(j * CHUNK, CHUNK)],
                    sem,
                )
            )
        for cp in copies:
            cp.wait()
        pltpu.sync_copy(
            rows_v, out_hbm.at[pl.ds(base + p * CPP * CHUNK, CPP * CHUNK)]
        )


def _sc_gather(table, idx3):
    kern = functools.partial(
        pl.kernel,
        mesh=plsc.VectorSubcoreMesh(core_axis_name="c", subcore_axis_name="s"),
        out_type=jax.ShapeDtypeStruct((N_IDX, 2 * E), jnp.float32),
        scratch_types=[
            pltpu.VMEM((NCHUNK, CHUNK), jnp.int32),
            pltpu.VMEM((CPP * CHUNK, 2 * E), jnp.float32),
            pltpu.SemaphoreType.DMA,
        ],
    )(_sc_gather_body)
    return kern(table, idx3)


def _lstm_body(x_ref, sel_ref, w_ref, b_ref, h0_ref, c0_ref,
               ys_ref, h_ref, c_ref):
    i = pl.program_id(0)

    @pl.when(i == 0)
    def _():
        h_ref[...] = h0_ref[...]
        c_ref[...] = c0_ref[...]

    h = h_ref[...]
    c = c_ref[...]
    b = b_ref[...]
    w = w_ref[...]
    for j in range(T_BLK):
        xf = x_ref[j * B:(j + 1) * B, :]
        sel = sel_ref[j * B:(j + 1) * B, :] != 0
        x_t = jnp.where(sel, xf[:, E:2 * E], xf[:, 0:E])
        xh = jnp.concatenate([x_t, h], axis=1)
        gates = jnp.dot(xh, w, preferred_element_type=jnp.float32) + b
        ig = jax.nn.sigmoid(gates[:, 0:H])
        fg = jax.nn.sigmoid(gates[:, H:2 * H])
        gg = jnp.tanh(gates[:, 2 * H:3 * H])
        og = jax.nn.sigmoid(gates[:, 3 * H:4 * H])
        c = fg * c + ig * gg
        h = og * jnp.tanh(c)
        ys_ref[:, j * H:(j + 1) * H] = h
    h_ref[...] = h
    c_ref[...] = c


def _lstm(x, sel2d, w_cat, bias, h0, c0, interpret=False):
    return pl.pallas_call(
        _lstm_body,
        grid=(N_BLK,),
        in_specs=[
            pl.BlockSpec((T_BLK * B, 2 * E), lambda i: (i, 0)),
            pl.BlockSpec((T_BLK * B, E), lambda i: (i, 0)),
            pl.BlockSpec((E + H, 4 * H), lambda i: (0, 0)),
            pl.BlockSpec((1, 4 * H), lambda i: (0, 0)),
            pl.BlockSpec((B, H), lambda i: (0, 0)),
            pl.BlockSpec((B, H), lambda i: (0, 0)),
        ],
        out_specs=[
            pl.BlockSpec((B, T_BLK * H), lambda i: (0, i)),
            pl.BlockSpec((B, H), lambda i: (0, 0)),
            pl.BlockSpec((B, H), lambda i: (0, 0)),
        ],
        out_shape=[
            jax.ShapeDtypeStruct((B, L * H), jnp.float32),
            jax.ShapeDtypeStruct((B, H), jnp.float32),
            jax.ShapeDtypeStruct((B, H), jnp.float32),
        ],
        compiler_params=pltpu.CompilerParams(
            dimension_semantics=("arbitrary",),
        ),
        interpret=interpret,
    )(x, sel2d, w_cat, bias, h0, c0)


def kernel(decoder_input, h0, c0, emb, W_ih, W_hh, b_ih, b_hh):
    idxT = decoder_input.T.reshape(-1).astype(jnp.int32)
    selT = ((idxT >> 14) & 1).astype(jnp.int8)
    idx_p = ((idxT >> 15) << 14) | (idxT & 16383)
    idx3 = idx_p.reshape(NW, NCHUNK, CHUNK)
    table = _pack(emb.T)
    x = _sc_gather(table, idx3)
    sel2d = jnp.broadcast_to(selT.reshape(N_IDX, 1), (N_IDX, E))
    w_cat = jnp.concatenate([W_ih.T, W_hh.T], axis=0)
    bias = (b_ih + b_hh).reshape(1, 4 * H)
    ys2d, h_n, c_n = _lstm(x, sel2d, w_cat, bias, h0[0], c0[0])
    decoder_output = ys2d.reshape(B, L, H)
    return decoder_output, (h_n[None, :, :], c_n[None, :, :])
